# baseline (speedup 1.0000x reference)
import functools

import jax
import jax.numpy as jnp
import numpy as np
from jax import lax
from jax.experimental import pallas as pl
from jax.experimental.pallas import tpu as pltpu
from jax.experimental.pallas import tpu_sc as plsc

N_STATES = 2097152
N_OBVS = 6
NC, NS, L = 2, 16, 16
NW = NC * NS
S_PER_W = N_STATES // NW
CH = 8192
N_CHUNKS = S_PER_W // CH
GROUPS = CH // L

_SPAT = np.array([[(16 * k + l) // 6 for l in range(L)] for k in range(6)],
                 dtype=np.int32)
_CPAT = np.array([[(16 * k + l) % 6 for l in range(L)] for k in range(6)],
                 dtype=np.int32)


def _sc_body(state_hbm, const_hbm, out_hbm, state_v, const_v, out_v):
    wid = lax.axis_index("s") * NC + lax.axis_index("c")
    pltpu.sync_copy(const_hbm, const_v)
    dvecs = [const_v[k] for k in range(6)]
    avec = const_v[6]
    nanv = const_v[7]
    for c in range(N_CHUNKS):
        base = wid * S_PER_W + c * CH
        pltpu.sync_copy(state_hbm.at[pl.ds(base, CH)], state_v)

        def body(g, carry):
            iota = lax.iota(jnp.int32, 16)
            for k in range(6):
                idx = 16 * g + (iota + 16 * k) // 6
                s6 = plsc.load_gather(state_v, [idx])
                val = jnp.where(s6 == 0, nanv,
                                jnp.where(s6 == 1, avec, dvecs[k]))
                out_v[pl.ds(96 * g + 16 * k, 16)] = val
            return carry

        lax.fori_loop(0, GROUPS, body, 0)
        pltpu.sync_copy(out_v, out_hbm.at[pl.ds(6 * base, 6 * CH)])


@functools.partial(jax.jit, static_argnames=())
def _expand(state, const):
    mesh = plsc.VectorSubcoreMesh(core_axis_name="c", subcore_axis_name="s",
                                  num_cores=NC, num_subcores=NS)
    f = pl.kernel(
        _sc_body,
        out_type=jax.ShapeDtypeStruct((N_STATES * N_OBVS,), jnp.float32),
        mesh=mesh,
        compiler_params=pltpu.CompilerParams(needs_layout_passes=False),
        scratch_types=[
            pltpu.VMEM((CH,), jnp.int32),
            pltpu.VMEM((8, L), jnp.float32),
            pltpu.VMEM((CH * 6,), jnp.float32),
        ],
    )
    return f(state, const)


def kernel(state, obvs, probs):
    lp = jnp.log(probs)
    dval = jnp.where(obvs == 6, lp[1], lp[2])
    const = jnp.concatenate([
        dval[jnp.asarray(_CPAT)].reshape(6, L),
        jnp.broadcast_to(lp[0], (1, L)),
        jnp.full((1, L), jnp.nan, jnp.float32),
    ], axis=0)
    out_flat = _expand(state, const)
    return out_flat.reshape(N_STATES, N_OBVS)


# speedup vs baseline: 1.0977x; 1.0977x over previous
import functools

import jax
import jax.numpy as jnp
import numpy as np
from jax import lax
from jax.experimental import pallas as pl
from jax.experimental.pallas import tpu as pltpu
from jax.experimental.pallas import tpu_sc as plsc

N_STATES = 2097152
N_OBVS = 6
NC, NS, L = 2, 16, 16
NW = NC * NS
S_PER_W = N_STATES // NW
CH = 8192
N_CHUNKS = S_PER_W // CH
GROUPS = CH // L

_CPAT = np.array([[(16 * k + l) % 6 for l in range(L)] for k in range(6)],
                 dtype=np.int32)


def _permute(x, idx):
    dnums = lax.GatherDimensionNumbers(
        offset_dims=(), collapsed_slice_dims=(0,), start_index_map=(0,))
    return lax.gather(x, idx[:, None], dnums, (1,),
                      mode=lax.GatherScatterMode.PROMISE_IN_BOUNDS)


def _sc_body(state_hbm, const_hbm, out_hbm,
             st_a, st_b, out_a, out_b, const_v,
             sin_a, sin_b, sout_a, sout_b):
    wid = lax.axis_index("s") * NC + lax.axis_index("c")
    pltpu.sync_copy(const_hbm, const_v)
    dvecs = [const_v[k] for k in range(6)]
    avec = const_v[6]
    nanv = const_v[7]

    stbufs = (st_a, st_b)
    outbufs = (out_a, out_b)
    sins = (sin_a, sin_b)
    souts = (sout_a, sout_b)
    w0 = wid * S_PER_W

    pltpu.async_copy(state_hbm.at[pl.ds(w0, CH)], st_a, sin_a)
    for c in range(N_CHUNKS):
        p = c % 2
        pltpu.make_async_copy(
            state_hbm.at[pl.ds(w0 + c * CH, CH)], stbufs[p], sins[p]).wait()
        if c + 1 < N_CHUNKS:
            pn = (c + 1) % 2
            pltpu.async_copy(
                state_hbm.at[pl.ds(w0 + (c + 1) * CH, CH)], stbufs[pn],
                sins[pn])
        if c >= 2:
            pltpu.make_async_copy(
                outbufs[p],
                out_hbm.at[pl.ds(6 * (w0 + (c - 2) * CH), 6 * CH)],
                souts[p]).wait()
        st = stbufs[p]
        ou = outbufs[p]
        iota = lax.iota(jnp.int32, 16)
        pats = [(iota + 16 * k) // 6 for k in range(6)]

        @plsc.parallel_loop(0, GROUPS)
        def body(g):
            s = st[pl.ds(16 * g, 16)]
            for k in range(6):
                s6 = _permute(s, pats[k])
                val = jnp.where(s6 == 0, nanv,
                                jnp.where(s6 == 1, avec, dvecs[k]))
                ou[pl.ds(96 * g + 16 * k, 16)] = val

        pltpu.async_copy(ou, out_hbm.at[pl.ds(6 * (w0 + c * CH), 6 * CH)],
                         souts[p])
    for c in (N_CHUNKS - 2, N_CHUNKS - 1):
        p = c % 2
        pltpu.make_async_copy(
            outbufs[p], out_hbm.at[pl.ds(6 * (w0 + c * CH), 6 * CH)],
            souts[p]).wait()


@jax.jit
def _expand(state, const):
    mesh = plsc.VectorSubcoreMesh(core_axis_name="c", subcore_axis_name="s",
                                  num_cores=NC, num_subcores=NS)
    f = pl.kernel(
        _sc_body,
        out_type=jax.ShapeDtypeStruct((N_STATES * N_OBVS,), jnp.float32),
        mesh=mesh,
        compiler_params=pltpu.CompilerParams(needs_layout_passes=False),
        scratch_types=[
            pltpu.VMEM((CH,), jnp.int32),
            pltpu.VMEM((CH,), jnp.int32),
            pltpu.VMEM((CH * 6,), jnp.float32),
            pltpu.VMEM((CH * 6,), jnp.float32),
            pltpu.VMEM((8, L), jnp.float32),
            pltpu.SemaphoreType.DMA,
            pltpu.SemaphoreType.DMA,
            pltpu.SemaphoreType.DMA,
            pltpu.SemaphoreType.DMA,
        ],
    )
    return f(state, const)


def kernel(state, obvs, probs):
    lp = jnp.log(probs)
    dval = jnp.where(obvs == 6, lp[1], lp[2])
    const = jnp.concatenate([
        dval[jnp.asarray(_CPAT)].reshape(6, L),
        jnp.broadcast_to(lp[0], (1, L)),
        jnp.full((1, L), jnp.nan, jnp.float32),
    ], axis=0)
    out_flat = _expand(state, const)
    return out_flat.reshape(N_STATES, N_OBVS)


# speedup vs baseline: 31.2877x; 28.5039x over previous
import jax
import jax.numpy as jnp
from jax import lax
from jax.experimental import pallas as pl
from jax.experimental.pallas import tpu as pltpu
from jax.experimental.pallas import tpu_sc as plsc

N_STATES = 2097152
N_OBVS = 6
NC, NS, L = 2, 16, 16
NW = NC * NS
S_PER_W = N_STATES // NW
CH = 4096
N_CHUNKS = S_PER_W // CH
GROUPS = CH // L


def _sc_body(state_hbm, const_hbm, out_hbm,
             st_a, st_b, out_a, out_b, const_v,
             sin_a, sin_b, sout_a, sout_b):
    wid = lax.axis_index("s") * NC + lax.axis_index("c")
    pltpu.sync_copy(const_hbm, const_v)
    dsplats = [const_v[j] for j in range(6)]
    avec = const_v[6]
    nanv = const_v[7]

    stbufs = (st_a, st_b)
    outbufs = (out_a, out_b)
    sins = (sin_a, sin_b)
    souts = (sout_a, sout_b)
    w0 = wid * S_PER_W

    pltpu.async_copy(state_hbm.at[pl.ds(w0, CH)], st_a, sin_a)
    for c in range(N_CHUNKS):
        p = c % 2
        base = w0 + c * CH
        pltpu.make_async_copy(
            state_hbm.at[pl.ds(base, CH)], stbufs[p], sins[p]).wait()
        if c + 1 < N_CHUNKS:
            pn = (c + 1) % 2
            pltpu.async_copy(
                state_hbm.at[pl.ds(base + CH, CH)], stbufs[pn], sins[pn])
        if c >= 2:
            pltpu.make_async_copy(
                outbufs[p],
                out_hbm.at[:, pl.ds(w0 + (c - 2) * CH, CH)],
                souts[p]).wait()
        st = stbufs[p]
        ou = outbufs[p]

        @plsc.parallel_loop(0, GROUPS)
        def body(g):
            s = st[pl.ds(16 * g, 16)]
            m0 = s == 0
            m1 = s == 1
            for j in range(6):
                ou[j, pl.ds(16 * g, 16)] = jnp.where(
                    m0, nanv, jnp.where(m1, avec, dsplats[j]))

        pltpu.async_copy(ou, out_hbm.at[:, pl.ds(base, CH)], souts[p])
    for c in (N_CHUNKS - 2, N_CHUNKS - 1):
        p = c % 2
        pltpu.make_async_copy(
            outbufs[p], out_hbm.at[:, pl.ds(w0 + c * CH, CH)],
            souts[p]).wait()


@jax.jit
def _expand(state, const):
    mesh = plsc.VectorSubcoreMesh(core_axis_name="c", subcore_axis_name="s",
                                  num_cores=NC, num_subcores=NS)
    f = pl.kernel(
        _sc_body,
        out_type=jax.ShapeDtypeStruct((N_OBVS, N_STATES), jnp.float32),
        mesh=mesh,
        compiler_params=pltpu.CompilerParams(needs_layout_passes=False,
                                             use_tc_tiling_on_sc=True),
        scratch_types=[
            pltpu.VMEM((CH,), jnp.int32),
            pltpu.VMEM((CH,), jnp.int32),
            pltpu.VMEM((N_OBVS, CH), jnp.float32),
            pltpu.VMEM((N_OBVS, CH), jnp.float32),
            pltpu.VMEM((8, L), jnp.float32),
            pltpu.SemaphoreType.DMA,
            pltpu.SemaphoreType.DMA,
            pltpu.SemaphoreType.DMA,
            pltpu.SemaphoreType.DMA,
        ],
    )
    return f(state, const)


def kernel(state, obvs, probs):
    lp = jnp.log(probs)
    dval = jnp.where(obvs == 6, lp[1], lp[2])
    const = jnp.concatenate([
        jnp.broadcast_to(dval[:, None], (6, L)),
        jnp.broadcast_to(lp[0], (1, L)),
        jnp.full((1, L), jnp.nan, jnp.float32),
    ], axis=0)
    out_cm = _expand(state, const)
    return out_cm.T
